# baseline (speedup 1.0000x reference)
import functools

import jax
import jax.numpy as jnp
from jax import lax
from jax.experimental import pallas as pl
from jax.experimental.pallas import tpu as pltpu
from jax.experimental.pallas import tpu_sc as plsc

D = 128
MAX_LEN = 300
LN_EPS = 1e-3

NC = 2
NS = 16
LANES = 16
NW = NC * NS

CHUNK = 64
GRP = 8
GSUB = CHUNK // LANES
DSUB = D // LANES


def _rsqrt_sigma(var):
    xc = jnp.maximum(var, 1e-30)
    xi = lax.bitcast_convert_type(xc, jnp.int32)
    yi = jnp.int32(0x5F3759DF) - (xi >> 1)
    y = lax.bitcast_convert_type(yi, jnp.float32)
    for _ in range(3):
        y = y * (1.5 - 0.5 * xc * y * y)
    return xc * y


_DNUMS = lax.GatherDimensionNumbers(
    offset_dims=(), collapsed_slice_dims=(0,), start_index_map=(0,))


def _hsum(v, lane):
    for k in (8, 4, 2, 1):
        perm = lax.bitwise_xor(lane, jnp.int32(k))
        v = v + lax.gather(
            v, perm[:, None], _DNUMS, slice_sizes=(1,),
            mode=lax.GatherScatterMode.PROMISE_IN_BOUNDS)
    return v


def _sc_body(emb0_hbm, emb1_hbm, post_hbm, xs0_hbm, xs1_hbm, ends_hbm,
             gain_hbm, bias_hbm,
             ann_out, cont_out, tim_out,
             idx0_all, idx1_all, idxp_all,
             rows0_v, rows1_v, rowsp_v, cont_v, ann_v,
             gain_v, bias_v, ends_v,
             sem_g0, sem_g1, sem_o0, sem_o1, sem_t0, sem_t1):
    wid = lax.axis_index("s") * NC + lax.axis_index("c")
    total = ann_out.shape[0]
    tokens_per_w = total // NW
    nchunks = tokens_per_w // CHUNK
    nseg = ends_v.shape[0]
    base_w = wid * tokens_per_w
    sem_g = (sem_g0, sem_g1)
    sem_o = (sem_o0, sem_o1)
    sem_t = (sem_t0, sem_t1)

    pltpu.sync_copy(gain_hbm, gain_v)
    pltpu.sync_copy(bias_hbm, bias_v)
    pltpu.sync_copy(ends_hbm, ends_v)
    pltpu.sync_copy(xs0_hbm.at[wid], idx0_all)
    pltpu.sync_copy(xs1_hbm.at[wid], idx1_all)

    lane = lax.iota(jnp.int32, LANES)
    gvecs = [gain_v[pl.ds(LANES * d, LANES)] for d in range(DSUB)]
    bvecs = [bias_v[pl.ds(LANES * d, LANES)] for d in range(DSUB)]

    def segscan(ci):
        base = base_w + ci * CHUNK
        gvs = [base + i * LANES + lane for i in range(GSUB)]

        def body(j, starts):
            ev = ends_v[pl.ds(j * LANES, LANES)]
            for l in range(LANES):
                e = ev[l]
                starts = tuple(
                    jnp.maximum(st, jnp.where(e <= g, e, 0))
                    for st, g in zip(starts, gvs))
            return starts

        starts = lax.fori_loop(
            0, nseg // LANES, body,
            tuple(jnp.zeros((LANES,), jnp.int32) for _ in range(GSUB)))
        pvs = [(gvs[i] - starts[i]) % MAX_LEN for i in range(GSUB)]
        for i in range(GSUB):
            idxp_all[ci, pl.ds(i * LANES, LANES)] = pvs[i]
        p0 = pvs[0][0]
        cnt = jnp.zeros((LANES,), jnp.int32)
        for i in range(GSUB):
            expect = p0 + i * LANES + lane
            cnt = cnt + jnp.where(pvs[i] == expect, 1, 0)
        islin = jnp.logical_and(
            _hsum(cnt, lane)[0] == CHUNK, p0 % 8 == 0)
        return p0, islin

    def issue_gathers(ci, p, p0, islin):
        pltpu.make_async_copy(
            emb0_hbm.at[idx0_all.at[ci]], rows0_v.at[p], sem_g[p]).start()
        pltpu.make_async_copy(
            emb1_hbm.at[idx1_all.at[ci]], rows1_v.at[p], sem_g[p]).start()

        @pl.when(islin)
        def _():
            pltpu.make_async_copy(
                post_hbm.at[pl.ds(pl.multiple_of(p0, 8), CHUNK)],
                rowsp_v.at[p], sem_g[p]).start()

        @pl.when(jnp.logical_not(islin))
        def _():
            pltpu.make_async_copy(
                post_hbm.at[idxp_all.at[ci]], rowsp_v.at[p],
                sem_g[p]).start()

    def wait_gathers(ci, p):
        pltpu.make_async_copy(
            emb0_hbm.at[idx0_all.at[ci]], rows0_v.at[p], sem_g[p]).wait()
        pltpu.make_async_copy(
            emb1_hbm.at[idx1_all.at[ci]], rows1_v.at[p], sem_g[p]).wait()
        pltpu.make_async_copy(
            post_hbm.at[pl.ds(0, CHUNK)], rowsp_v.at[p], sem_g[p]).wait()

    def out_copies(ci, p):
        base = base_w + ci * CHUNK
        return (
            pltpu.make_async_copy(
                cont_v.at[p], cont_out.at[pl.ds(base, CHUNK)], sem_o[p]),
            pltpu.make_async_copy(
                ann_v.at[p], ann_out.at[pl.ds(base, CHUNK)], sem_o[p]),
        )

    def tim_copy(ci, p):
        base = base_w + ci * CHUNK
        return pltpu.make_async_copy(
            rowsp_v.at[p], tim_out.at[pl.ds(base, CHUNK)], sem_t[p])

    def compute(ci, p):
        def grp_body(q, _):
            for l in range(GRP):
                t = q * GRP + l
                avs = []
                s = jnp.zeros((LANES,), jnp.float32)
                sq = jnp.zeros((LANES,), jnp.float32)
                for d in range(DSUB):
                    sl = pl.ds(LANES * d, LANES)
                    c = rows0_v[p, t, sl] + rows1_v[p, t, sl]
                    cont_v[p, t, sl] = c
                    a = c + rowsp_v[p, t, sl]
                    avs.append(a)
                    s = s + a
                    sq = sq + a * a
                mu = _hsum(s, lane) * (1.0 / D)
                var = _hsum(sq, lane) * (1.0 / D) - mu * mu
                r = 1.0 / (_rsqrt_sigma(var) + LN_EPS)
                for d in range(DSUB):
                    sl = pl.ds(LANES * d, LANES)
                    ann_v[p, t, sl] = (avs[d] - mu) * r * gvecs[d] + bvecs[d]
            return 0

        lax.fori_loop(0, CHUNK // GRP, grp_body, 0)

    p0_0, islin_0 = segscan(0)
    issue_gathers(0, 0, p0_0, islin_0)

    def superstep(s, _):
        for p in (0, 1):
            ci = 2 * s + p
            nxt = ci + 1
            pn = 1 - p

            def prep():
                p0n, islinn = segscan(nxt)
                issue_gathers(nxt, pn, p0n, islinn)

            if p == 0:
                @pl.when(s >= 1)
                def _():
                    tim_copy(ci - 1, pn).wait()
                prep()
            else:
                @pl.when(s < (nchunks // 2) - 1)
                def _():
                    tim_copy(ci - 1, pn).wait()
                    prep()

            wait_gathers(ci, p)

            @pl.when(s >= 1)
            def _():
                for cp in out_copies(ci - 2, p):
                    cp.wait()

            compute(ci, p)
            for cp in out_copies(ci, p):
                cp.start()
            tim_copy(ci, p).start()
        return 0

    lax.fori_loop(0, nchunks // 2, superstep, 0)

    for p in (0, 1):
        ci = nchunks - 2 + p
        for cp in out_copies(ci, p):
            cp.wait()
        tim_copy(ci, p).wait()


def kernel(xs_0, xs_1, pre_words_idxs, batch_idxs_seq_lens, emb0, emb1,
           position_table, ln_gain, ln_bias):
    del pre_words_idxs
    T = xs_0.shape[0]
    tokens_per_w = T // NW
    nchunks = tokens_per_w // CHUNK
    assert T % (NW * CHUNK) == 0 and nchunks % 2 == 0
    xs_0 = xs_0.astype(jnp.int32).reshape(NW, nchunks, CHUNK)
    xs_1 = xs_1.astype(jnp.int32).reshape(NW, nchunks, CHUNK)
    ends = jnp.cumsum(batch_idxs_seq_lens.astype(jnp.int32))

    n1, npos = emb1.shape[0], position_table.shape[0]
    out_sd = jax.ShapeDtypeStruct((T, D), jnp.float32)
    mesh = plsc.VectorSubcoreMesh(
        core_axis_name="c", subcore_axis_name="s", num_cores=NC,
        num_subcores=NS)
    run = pl.kernel(
        _sc_body,
        out_type=(out_sd, out_sd, out_sd),
        mesh=mesh,
        scratch_types=[
            pltpu.VMEM((nchunks, CHUNK), jnp.int32),
            pltpu.VMEM((nchunks, CHUNK), jnp.int32),
            pltpu.VMEM((nchunks, CHUNK), jnp.int32),
            pltpu.VMEM((2, CHUNK, D), jnp.float32),
            pltpu.VMEM((2, CHUNK, D), jnp.float32),
            pltpu.VMEM((2, CHUNK, D), jnp.float32),
            pltpu.VMEM((2, CHUNK, D), jnp.float32),
            pltpu.VMEM((2, CHUNK, D), jnp.float32),
            pltpu.VMEM((D,), jnp.float32),
            pltpu.VMEM((D,), jnp.float32),
            pltpu.VMEM((ends.shape[0],), jnp.int32),
            pltpu.SemaphoreType.DMA,
            pltpu.SemaphoreType.DMA,
            pltpu.SemaphoreType.DMA,
            pltpu.SemaphoreType.DMA,
            pltpu.SemaphoreType.DMA,
            pltpu.SemaphoreType.DMA,
        ],
    )
    annotations, content, timing = run(
        emb0, emb1, position_table, xs_0, xs_1, ends, ln_gain, ln_bias)
    return (annotations, content, timing)


# speedup vs baseline: 1.0120x; 1.0120x over previous
import functools

import jax
import jax.numpy as jnp
from jax import lax
from jax.experimental import pallas as pl
from jax.experimental.pallas import tpu as pltpu
from jax.experimental.pallas import tpu_sc as plsc

D = 128
MAX_LEN = 300
LN_EPS = 1e-3

NC = 2
NS = 16
LANES = 16
NW = NC * NS

CHUNK = 64
GRP = 8
GSUB = CHUNK // LANES
DSUB = D // LANES


def _rsqrt_sigma(var):
    xc = jnp.maximum(var, 1e-30)
    xi = lax.bitcast_convert_type(xc, jnp.int32)
    yi = jnp.int32(0x5F3759DF) - (xi >> 1)
    y = lax.bitcast_convert_type(yi, jnp.float32)
    for _ in range(3):
        y = y * (1.5 - 0.5 * xc * y * y)
    return xc * y


_DNUMS = lax.GatherDimensionNumbers(
    offset_dims=(), collapsed_slice_dims=(0,), start_index_map=(0,))


def _hsum(v, lane):
    for k in (8, 4, 2, 1):
        perm = lax.bitwise_xor(lane, jnp.int32(k))
        v = v + lax.gather(
            v, perm[:, None], _DNUMS, slice_sizes=(1,),
            mode=lax.GatherScatterMode.PROMISE_IN_BOUNDS)
    return v


def _sc_body(emb0_hbm, emb1_hbm, post_hbm, xs0_hbm, xs1_hbm, ends_hbm,
             gain_hbm, bias_hbm,
             ann_out, cont_out, tim_out,
             idx0_all, idx1_all, idxp_all,
             rows0_v, rows1_v, rowsp_v, cont_v, ann_v,
             gain_v, bias_v, ends_v,
             sem_g0, sem_g1, sem_o0, sem_o1, sem_t0, sem_t1):
    wid = lax.axis_index("s") * NC + lax.axis_index("c")
    total = ann_out.shape[0]
    tokens_per_w = total // NW
    nchunks = tokens_per_w // CHUNK
    nseg = ends_v.shape[0]
    base_w = wid * tokens_per_w
    sem_g = (sem_g0, sem_g1)
    sem_o = (sem_o0, sem_o1)
    sem_t = (sem_t0, sem_t1)

    _stage = (
        (gain_hbm, gain_v), (bias_hbm, bias_v), (ends_hbm, ends_v),
        (xs0_hbm.at[wid], idx0_all), (xs1_hbm.at[wid], idx1_all))
    for src, dst in _stage:
        pltpu.make_async_copy(src, dst, sem_g0).start()
    for src, dst in _stage:
        pltpu.make_async_copy(src, dst, sem_g0).wait()

    lane = lax.iota(jnp.int32, LANES)
    gvecs = [gain_v[pl.ds(LANES * d, LANES)] for d in range(DSUB)]
    bvecs = [bias_v[pl.ds(LANES * d, LANES)] for d in range(DSUB)]

    def segscan(ci):
        base = base_w + ci * CHUNK
        gvs = [base + i * LANES + lane for i in range(GSUB)]

        def body(j, starts):
            ev = ends_v[pl.ds(j * LANES, LANES)]
            for l in range(LANES):
                e = ev[l]
                starts = tuple(
                    jnp.maximum(st, jnp.where(e <= g, e, 0))
                    for st, g in zip(starts, gvs))
            return starts

        starts = lax.fori_loop(
            0, nseg // LANES, body,
            tuple(jnp.zeros((LANES,), jnp.int32) for _ in range(GSUB)))
        pvs = [(gvs[i] - starts[i]) % MAX_LEN for i in range(GSUB)]
        for i in range(GSUB):
            idxp_all[ci, pl.ds(i * LANES, LANES)] = pvs[i]
        p0 = pvs[0][0]
        cnt = jnp.zeros((LANES,), jnp.int32)
        for i in range(GSUB):
            expect = p0 + i * LANES + lane
            cnt = cnt + jnp.where(pvs[i] == expect, 1, 0)
        islin = jnp.logical_and(
            _hsum(cnt, lane)[0] == CHUNK, p0 % 8 == 0)
        return p0, islin

    def issue_gathers(ci, p, p0, islin):
        pltpu.make_async_copy(
            emb0_hbm.at[idx0_all.at[ci]], rows0_v.at[p], sem_g[p]).start()
        pltpu.make_async_copy(
            emb1_hbm.at[idx1_all.at[ci]], rows1_v.at[p], sem_g[p]).start()

        @pl.when(islin)
        def _():
            pltpu.make_async_copy(
                post_hbm.at[pl.ds(pl.multiple_of(p0, 8), CHUNK)],
                rowsp_v.at[p], sem_g[p]).start()

        @pl.when(jnp.logical_not(islin))
        def _():
            pltpu.make_async_copy(
                post_hbm.at[idxp_all.at[ci]], rowsp_v.at[p],
                sem_g[p]).start()

    def wait_gathers(ci, p):
        pltpu.make_async_copy(
            emb0_hbm.at[idx0_all.at[ci]], rows0_v.at[p], sem_g[p]).wait()
        pltpu.make_async_copy(
            emb1_hbm.at[idx1_all.at[ci]], rows1_v.at[p], sem_g[p]).wait()
        pltpu.make_async_copy(
            post_hbm.at[pl.ds(0, CHUNK)], rowsp_v.at[p], sem_g[p]).wait()

    def out_copies(ci, p):
        base = base_w + ci * CHUNK
        return (
            pltpu.make_async_copy(
                cont_v.at[p], cont_out.at[pl.ds(base, CHUNK)], sem_o[p]),
            pltpu.make_async_copy(
                ann_v.at[p], ann_out.at[pl.ds(base, CHUNK)], sem_o[p]),
        )

    def tim_copy(ci, p):
        base = base_w + ci * CHUNK
        return pltpu.make_async_copy(
            rowsp_v.at[p], tim_out.at[pl.ds(base, CHUNK)], sem_t[p])

    def compute(ci, p):
        def grp_body(q, _):
            for l in range(GRP):
                t = q * GRP + l
                avs = []
                s = jnp.zeros((LANES,), jnp.float32)
                sq = jnp.zeros((LANES,), jnp.float32)
                for d in range(DSUB):
                    sl = pl.ds(LANES * d, LANES)
                    c = rows0_v[p, t, sl] + rows1_v[p, t, sl]
                    cont_v[p, t, sl] = c
                    a = c + rowsp_v[p, t, sl]
                    avs.append(a)
                    s = s + a
                    sq = sq + a * a
                mu = _hsum(s, lane) * (1.0 / D)
                var = _hsum(sq, lane) * (1.0 / D) - mu * mu
                r = 1.0 / (_rsqrt_sigma(var) + LN_EPS)
                for d in range(DSUB):
                    sl = pl.ds(LANES * d, LANES)
                    ann_v[p, t, sl] = (avs[d] - mu) * r * gvecs[d] + bvecs[d]
            return 0

        lax.fori_loop(0, CHUNK // GRP, grp_body, 0)

    p0_0, islin_0 = segscan(0)
    issue_gathers(0, 0, p0_0, islin_0)

    def superstep(s, _):
        for p in (0, 1):
            ci = 2 * s + p
            nxt = ci + 1
            pn = 1 - p

            def prep():
                p0n, islinn = segscan(nxt)
                issue_gathers(nxt, pn, p0n, islinn)

            if p == 0:
                @pl.when(s >= 1)
                def _():
                    tim_copy(ci - 1, pn).wait()
                prep()
            else:
                @pl.when(s < (nchunks // 2) - 1)
                def _():
                    tim_copy(ci - 1, pn).wait()
                    prep()

            wait_gathers(ci, p)

            @pl.when(s >= 1)
            def _():
                for cp in out_copies(ci - 2, p):
                    cp.wait()

            compute(ci, p)
            for cp in out_copies(ci, p):
                cp.start()
            tim_copy(ci, p).start()
        return 0

    lax.fori_loop(0, nchunks // 2, superstep, 0)

    for p in (0, 1):
        ci = nchunks - 2 + p
        for cp in out_copies(ci, p):
            cp.wait()
        tim_copy(ci, p).wait()


def kernel(xs_0, xs_1, pre_words_idxs, batch_idxs_seq_lens, emb0, emb1,
           position_table, ln_gain, ln_bias):
    del pre_words_idxs
    T = xs_0.shape[0]
    tokens_per_w = T // NW
    nchunks = tokens_per_w // CHUNK
    assert T % (NW * CHUNK) == 0 and nchunks % 2 == 0
    xs_0 = xs_0.astype(jnp.int32).reshape(NW, nchunks, CHUNK)
    xs_1 = xs_1.astype(jnp.int32).reshape(NW, nchunks, CHUNK)
    ends = jnp.cumsum(batch_idxs_seq_lens.astype(jnp.int32))

    n1, npos = emb1.shape[0], position_table.shape[0]
    out_sd = jax.ShapeDtypeStruct((T, D), jnp.float32)
    mesh = plsc.VectorSubcoreMesh(
        core_axis_name="c", subcore_axis_name="s", num_cores=NC,
        num_subcores=NS)
    run = pl.kernel(
        _sc_body,
        out_type=(out_sd, out_sd, out_sd),
        mesh=mesh,
        scratch_types=[
            pltpu.VMEM((nchunks, CHUNK), jnp.int32),
            pltpu.VMEM((nchunks, CHUNK), jnp.int32),
            pltpu.VMEM((nchunks, CHUNK), jnp.int32),
            pltpu.VMEM((2, CHUNK, D), jnp.float32),
            pltpu.VMEM((2, CHUNK, D), jnp.float32),
            pltpu.VMEM((2, CHUNK, D), jnp.float32),
            pltpu.VMEM((2, CHUNK, D), jnp.float32),
            pltpu.VMEM((2, CHUNK, D), jnp.float32),
            pltpu.VMEM((D,), jnp.float32),
            pltpu.VMEM((D,), jnp.float32),
            pltpu.VMEM((ends.shape[0],), jnp.int32),
            pltpu.SemaphoreType.DMA,
            pltpu.SemaphoreType.DMA,
            pltpu.SemaphoreType.DMA,
            pltpu.SemaphoreType.DMA,
            pltpu.SemaphoreType.DMA,
            pltpu.SemaphoreType.DMA,
        ],
    )
    annotations, content, timing = run(
        emb0, emb1, position_table, xs_0, xs_1, ends, ln_gain, ln_bias)
    return (annotations, content, timing)
